# baseline (speedup 1.0000x reference)
import functools

import jax
import jax.numpy as jnp
from jax import lax
from jax.experimental import pallas as pl
from jax.experimental.pallas import tpu as pltpu
from jax.experimental.pallas import tpu_sc as plsc

N = 10000
E = 320000
T = 2
D = 128

NP = 10240
NUM_TILES = 16
LANES = 16
EDGE_BLK = 80
GRAN = 4 * EDGE_BLK
E_PER_TILE = E // NUM_TILES
NBLK = E_PER_TILE // EDGE_BLK
NQUAD = NBLK // 4
ROWS_PER_TILE = NP // NUM_TILES
EPAD = GRAN // 2


def _sc_body(stride, x_hbm, src_hbm, dst_hbm, w_hbm, out0_hbm, out1_hbm,
             acc, rows0, rows1, rows2, rows3, dx0, dx1, dx2, dx3,
             sxq0, sxq1, wvq0, wvq1,
             gsem0, gsem1, gsem2, gsem3, ssem0, ssem1, ssem2, ssem3,
             dsem0, dsem1, dsem2, dsem3,
             xsem0, xsem1, wsem0, wsem1, zsem):
    t = lax.axis_index("c")
    s = lax.axis_index("s")
    ebase = s * E_PER_TILE
    row0 = s * ROWS_PER_TILE
    toff = t * stride

    rows = (rows0, rows1, rows2, rows3)
    dxb = (dx0, dx1, dx2, dx3)
    sxq = (sxq0, sxq1)
    wvq = (wvq0, wvq1)
    gsem = (gsem0, gsem1, gsem2, gsem3)
    ssem = (ssem0, ssem1, ssem2, ssem3)
    dsem = (dsem0, dsem1, dsem2, dsem3)
    xsem = (xsem0, xsem1)
    wsem = (wsem0, wsem1)

    def _zrow(r, c):
        for d in range(D // LANES):
            rows2[r, pl.ds(d * LANES, LANES)] = jnp.zeros((LANES,),
                                                          jnp.float32)
        return c
    lax.fori_loop(0, EDGE_BLK, _zrow, 0)
    for z in range(ROWS_PER_TILE // EDGE_BLK):
        pltpu.async_copy(rows2, acc.at[pl.ds(row0 + z * EDGE_BLK, EDGE_BLK)],
                         zsem)

    def didx_load(b, r4, p=0):
        pltpu.async_copy(dst_hbm.at[pl.ds(ebase + b * EDGE_BLK, EDGE_BLK)],
                         dxb[r4], dsem[r4])

    def gran_load(g, P):
        sl = pl.ds(ebase + g * GRAN, GRAN)
        pltpu.async_copy(src_hbm.at[sl], sxq[P], xsem[P])
        pltpu.async_copy(w_hbm.at[sl], wvq[P], wsem[P])

    def wait_blk(sem, dst):
        pltpu.make_async_copy(dst_hbm.at[pl.ds(0, EDGE_BLK)], dst, sem).wait()

    def wait_gran(P):
        pltpu.make_async_copy(src_hbm.at[pl.ds(0, GRAN)], sxq[P],
                              xsem[P]).wait()
        pltpu.make_async_copy(w_hbm.at[pl.ds(0, GRAN)], wvq[P],
                              wsem[P]).wait()
        def _off(i, c):
            sl = pl.ds(i * LANES, LANES)
            sxq[P][sl] = sxq[P][sl] + toff
            return c
        lax.fori_loop(0, GRAN // LANES, _off, 0)

    def wait_rows(sem, dst):
        pltpu.make_async_copy(x_hbm.at[pl.ds(0, EDGE_BLK)], dst, sem).wait()

    def gather(xP, xoff, r4, p=0):
        idx = sxq[xP].at[pl.ds(xoff, EDGE_BLK)]
        pltpu.async_copy(x_hbm.at[idx], rows[r4], gsem[r4])

    def step(b, r4, p, wP, woff, first, last, xP=0, xoff=0):
        r4n = (r4 + 2) % 4
        if not first:
            wait_rows(ssem[r4n], rows[r4n])
        if not last:
            didx_load(b + 2, r4n)
            gather(xP, xoff, r4n)
        wait_rows(gsem[r4], rows[r4])
        wait_blk(dsem[r4], dxb[r4])

        def _sg(g, c):
            w16 = wvq[wP][pl.ds(woff + g * LANES, LANES)]
            for j in range(LANES):
                e = g * LANES + j
                w_e = w16[j]
                for d in range(D // LANES):
                    sl = pl.ds(d * LANES, LANES)
                    rows[r4][e, sl] = rows[r4][e, sl] * w_e
            return c
        lax.fori_loop(0, EDGE_BLK // LANES, _sg, 0)

        pltpu.async_copy(rows[r4], acc.at[dxb[r4]], ssem[r4], add=True)

    didx_load(0, 0, 0)
    didx_load(1, 1, 1)
    gran_load(0, 0)
    wait_gran(0)
    gran_load(1, 1)
    gather(0, 0 * EDGE_BLK, 0, 0)
    gather(0, 1 * EDGE_BLK, 1, 1)
    for z in range(ROWS_PER_TILE // EDGE_BLK):
        pltpu.make_async_copy(x_hbm.at[pl.ds(0, EDGE_BLK)], rows2,
                              zsem).wait()
    plsc.subcore_barrier()
    step(0, 0, 0, 0, 0 * EDGE_BLK, True, False, 0, 2 * EDGE_BLK)
    step(1, 1, 1, 0, 1 * EDGE_BLK, True, False, 0, 3 * EDGE_BLK)
    wait_gran(1)
    step(2, 2, 0, 0, 2 * EDGE_BLK, False, False, 1, 0)
    step(3, 3, 1, 0, 3 * EDGE_BLK, False, False, 1, EDGE_BLK)
    gran_load(2, 0)

    def _pair(i, c):
        b0 = 8 * i + 4
        step(b0 + 0, 0, 0, 1, 0 * EDGE_BLK, False, False, 1, 2 * EDGE_BLK)
        step(b0 + 1, 1, 1, 1, 1 * EDGE_BLK, False, False, 1, 3 * EDGE_BLK)
        wait_gran(0)
        step(b0 + 2, 2, 0, 1, 2 * EDGE_BLK, False, False, 0, 0)
        step(b0 + 3, 3, 1, 1, 3 * EDGE_BLK, False, False, 0, EDGE_BLK)
        gran_load_dyn(i, 1)
        step(b0 + 4, 0, 0, 0, 0 * EDGE_BLK, False, False, 0, 2 * EDGE_BLK)
        step(b0 + 5, 1, 1, 0, 1 * EDGE_BLK, False, False, 0, 3 * EDGE_BLK)
        wait_gran(1)
        step(b0 + 6, 2, 0, 0, 2 * EDGE_BLK, False, False, 1, 0)
        step(b0 + 7, 3, 1, 0, 3 * EDGE_BLK, False, False, 1, EDGE_BLK)
        gran_load_dyn2(i, 0)
        return c

    def gran_load_dyn(i, P):
        sl = pl.ds(ebase + (8 * i + 12) * EDGE_BLK, GRAN)
        pltpu.async_copy(src_hbm.at[sl], sxq[P], xsem[P])
        pltpu.async_copy(w_hbm.at[sl], wvq[P], wsem[P])

    def gran_load_dyn2(i, P):
        sl = pl.ds(ebase + (8 * i + 16) * EDGE_BLK, GRAN)
        pltpu.async_copy(src_hbm.at[sl], sxq[P], xsem[P])
        pltpu.async_copy(w_hbm.at[sl], wvq[P], wsem[P])

    lax.fori_loop(0, (NQUAD - 2) // 2, _pair, 0)

    b0 = 4 * (NQUAD - 1)
    step(b0 + 0, 0, 0, 1, 0 * EDGE_BLK, False, False, 1, 2 * EDGE_BLK)
    step(b0 + 1, 1, 1, 1, 1 * EDGE_BLK, False, False, 1, 3 * EDGE_BLK)
    wait_gran(0)
    step(b0 + 2, 2, 0, 1, 2 * EDGE_BLK, False, False, 0, 0)
    step(b0 + 3, 3, 1, 1, 3 * EDGE_BLK, False, False, 0, EDGE_BLK)
    step(NBLK - 2, 0, 0, 0, 0 * EDGE_BLK, False, True)
    step(NBLK - 1, 1, 1, 0, 1 * EDGE_BLK, False, True)

    wait_rows(ssem[0], rows0)
    wait_rows(ssem[1], rows1)

    plsc.subcore_barrier()

    @pl.when(t == 0)
    def _():
        pltpu.sync_copy(acc.at[pl.ds(row0, ROWS_PER_TILE)],
                        out0_hbm.at[pl.ds(row0, ROWS_PER_TILE)])

    @pl.when(t != 0)
    def _():
        pltpu.sync_copy(acc.at[pl.ds(row0, ROWS_PER_TILE)],
                        out1_hbm.at[pl.ds(row0, ROWS_PER_TILE)])


@functools.partial(jax.jit, static_argnums=0)
def _sc_propagate(stride, x2n, srcp, dst, wp):
    mesh = plsc.VectorSubcoreMesh(core_axis_name="c", subcore_axis_name="s")
    f = pl.kernel(
        functools.partial(_sc_body, stride),
        out_type=(jax.ShapeDtypeStruct((NP, D), jnp.float32),
                  jax.ShapeDtypeStruct((NP, D), jnp.float32)),
        mesh=mesh,
        scratch_types=(
            [pltpu.VMEM_SHARED((NP, D), jnp.float32)]
            + [pltpu.VMEM((EDGE_BLK, D), jnp.float32) for _ in range(4)]
            + [pltpu.VMEM((EDGE_BLK,), jnp.int32) for _ in range(4)]
            + [pltpu.VMEM((GRAN,), jnp.int32) for _ in range(2)]
            + [pltpu.VMEM((GRAN,), jnp.float32) for _ in range(2)]
            + [pltpu.SemaphoreType.DMA for _ in range(17)]
        ),
    )
    return f(x2n, srcp, dst, wp)


def _tc_mid_body(x0_ref, x1_ref, w_ref, o_ref):
    c = pl.program_id(0)
    w = w_ref[...]

    @pl.when(c == 0)
    def _():
        o_ref[...] = jnp.maximum(
            jnp.dot(x0_ref[...], w, preferred_element_type=jnp.float32), 0.0)

    @pl.when(c != 0)
    def _():
        o_ref[...] = jnp.maximum(
            jnp.dot(x1_ref[...], w, preferred_element_type=jnp.float32), 0.0)


@jax.jit
def _tc_mid(a0, a1, W):
    bn = 2048
    nb = NP // bn
    return pl.pallas_call(
        _tc_mid_body,
        grid=(T, nb),
        in_specs=[
            pl.BlockSpec((bn, D), lambda c, j: (j, 0)),
            pl.BlockSpec((bn, D), lambda c, j: (j, 0)),
            pl.BlockSpec((D, D), lambda c, j: (0, 0)),
        ],
        out_specs=pl.BlockSpec((bn, D), lambda c, j: (c * (NP // 2048) + j, 0)),
        out_shape=jax.ShapeDtypeStruct((T * NP, D), jnp.float32),
    )(a0, a1, W)


def _tc_final_body(x0_ref, x1_ref, w_ref, o_ref):
    w = w_ref[...]
    o_ref[0] = jnp.maximum(
        jnp.dot(x0_ref[...], w, preferred_element_type=jnp.float32), 0.0)
    o_ref[1] = jnp.maximum(
        jnp.dot(x1_ref[...], w, preferred_element_type=jnp.float32), 0.0)


@jax.jit
def _tc_final(a0, a1, W):
    bn = 2000
    return pl.pallas_call(
        _tc_final_body,
        grid=(N // bn,),
        in_specs=[
            pl.BlockSpec((bn, D), lambda i: (i, 0)),
            pl.BlockSpec((bn, D), lambda i: (i, 0)),
            pl.BlockSpec((D, D), lambda i: (0, 0)),
        ],
        out_specs=pl.BlockSpec((T, bn, D), lambda i: (0, i, 0)),
        out_shape=jax.ShapeDtypeStruct((T, N, D), jnp.float32),
    )(a0, a1, W)


def kernel(node_embs, edge_index, edge_weight, W1, W2):
    src = edge_index[0]
    dst = edge_index[1]
    pad_i = jnp.zeros((EPAD,), jnp.int32)
    pad_f = jnp.zeros((EPAD,), jnp.float32)
    srcp = jnp.concatenate([src, pad_i])
    wp = jnp.concatenate([edge_weight, pad_f])
    x = node_embs.reshape(T * N, D)
    a10, a11 = _sc_propagate(N, x, srcp, dst, wp)
    h1 = _tc_mid(a10, a11, W1)
    a20, a21 = _sc_propagate(NP, h1, srcp, dst, wp)
    return _tc_final(a20, a21, W2)


# speedup vs baseline: 1.0196x; 1.0196x over previous
import functools

import jax
import jax.numpy as jnp
from jax import lax
from jax.experimental import pallas as pl
from jax.experimental.pallas import tpu as pltpu
from jax.experimental.pallas import tpu_sc as plsc

N = 10000
E = 320000
T = 2
D = 128

NP = 10240
NUM_TILES = 16
LANES = 16
EDGE_BLK = 80
GRAN = 4 * EDGE_BLK
E_PER_TILE = E // NUM_TILES
NBLK = E_PER_TILE // EDGE_BLK
NQUAD = NBLK // 4
ROWS_PER_TILE = NP // NUM_TILES
EPAD = GRAN // 2


def _sc_body(x0_hbm, x1_hbm, src_hbm, dst_hbm, w_hbm, out0_hbm, out1_hbm,
             acc, rows0, rows1, rows2, rows3, dx0, dx1, dx2, dx3,
             sxq0, sxq1, wvq0, wvq1,
             gsem0, gsem1, gsem2, gsem3, ssem0, ssem1, ssem2, ssem3,
             dsem0, dsem1, dsem2, dsem3,
             xsem0, xsem1, wsem0, wsem1, zsem):
    t = lax.axis_index("c")
    s = lax.axis_index("s")
    ebase = s * E_PER_TILE
    row0 = s * ROWS_PER_TILE

    rows = (rows0, rows1, rows2, rows3)
    dxb = (dx0, dx1, dx2, dx3)
    sxq = (sxq0, sxq1)
    wvq = (wvq0, wvq1)
    gsem = (gsem0, gsem1, gsem2, gsem3)
    ssem = (ssem0, ssem1, ssem2, ssem3)
    dsem = (dsem0, dsem1, dsem2, dsem3)
    xsem = (xsem0, xsem1)
    wsem = (wsem0, wsem1)

    def _zrow(r, c):
        for d in range(D // LANES):
            rows2[r, pl.ds(d * LANES, LANES)] = jnp.zeros((LANES,),
                                                          jnp.float32)
        return c
    lax.fori_loop(0, EDGE_BLK, _zrow, 0)
    for z in range(ROWS_PER_TILE // EDGE_BLK):
        pltpu.async_copy(rows2, acc.at[pl.ds(row0 + z * EDGE_BLK, EDGE_BLK)],
                         zsem)

    def didx_load(b, r4, p=0):
        pltpu.async_copy(dst_hbm.at[pl.ds(ebase + b * EDGE_BLK, EDGE_BLK)],
                         dxb[r4], dsem[r4])

    def gran_load(g, P):
        sl = pl.ds(ebase + g * GRAN, GRAN)
        pltpu.async_copy(src_hbm.at[sl], sxq[P], xsem[P])
        pltpu.async_copy(w_hbm.at[sl], wvq[P], wsem[P])

    def wait_blk(sem, dst):
        pltpu.make_async_copy(dst_hbm.at[pl.ds(0, EDGE_BLK)], dst, sem).wait()

    def wait_gran(P):
        pltpu.make_async_copy(src_hbm.at[pl.ds(0, GRAN)], sxq[P],
                              xsem[P]).wait()
        pltpu.make_async_copy(w_hbm.at[pl.ds(0, GRAN)], wvq[P],
                              wsem[P]).wait()

    def wait_rows(sem, dst):
        pltpu.make_async_copy(x0_hbm.at[pl.ds(0, EDGE_BLK)], dst, sem).wait()

    def gather(xP, xoff, r4, p=0):
        idx = sxq[xP].at[pl.ds(xoff, EDGE_BLK)]

        @pl.when(t == 0)
        def _():
            pltpu.async_copy(x0_hbm.at[idx], rows[r4], gsem[r4])

        @pl.when(t != 0)
        def _():
            pltpu.async_copy(x1_hbm.at[idx], rows[r4], gsem[r4])

    def step(b, r4, p, wP, woff, first, last, xP=0, xoff=0):
        r4n = (r4 + 2) % 4
        if not first:
            wait_rows(ssem[r4n], rows[r4n])
        if not last:
            didx_load(b + 2, r4n)
            gather(xP, xoff, r4n)
        wait_rows(gsem[r4], rows[r4])
        wait_blk(dsem[r4], dxb[r4])

        def _sg(g, c):
            w16 = wvq[wP][pl.ds(woff + g * LANES, LANES)]
            for j in range(LANES):
                e = g * LANES + j
                w_e = w16[j]
                for d in range(D // LANES):
                    sl = pl.ds(d * LANES, LANES)
                    rows[r4][e, sl] = rows[r4][e, sl] * w_e
            return c
        lax.fori_loop(0, EDGE_BLK // LANES, _sg, 0)

        pltpu.async_copy(rows[r4], acc.at[dxb[r4]], ssem[r4], add=True)

    didx_load(0, 0, 0)
    didx_load(1, 1, 1)
    gran_load(0, 0)
    wait_gran(0)
    gran_load(1, 1)
    gather(0, 0 * EDGE_BLK, 0, 0)
    gather(0, 1 * EDGE_BLK, 1, 1)
    for z in range(ROWS_PER_TILE // EDGE_BLK):
        pltpu.make_async_copy(x0_hbm.at[pl.ds(0, EDGE_BLK)], rows2,
                              zsem).wait()
    plsc.subcore_barrier()
    step(0, 0, 0, 0, 0 * EDGE_BLK, True, False, 0, 2 * EDGE_BLK)
    step(1, 1, 1, 0, 1 * EDGE_BLK, True, False, 0, 3 * EDGE_BLK)
    wait_gran(1)
    step(2, 2, 0, 0, 2 * EDGE_BLK, False, False, 1, 0)
    step(3, 3, 1, 0, 3 * EDGE_BLK, False, False, 1, EDGE_BLK)
    gran_load(2, 0)

    def _pair(i, c):
        b0 = 8 * i + 4
        step(b0 + 0, 0, 0, 1, 0 * EDGE_BLK, False, False, 1, 2 * EDGE_BLK)
        step(b0 + 1, 1, 1, 1, 1 * EDGE_BLK, False, False, 1, 3 * EDGE_BLK)
        wait_gran(0)
        step(b0 + 2, 2, 0, 1, 2 * EDGE_BLK, False, False, 0, 0)
        step(b0 + 3, 3, 1, 1, 3 * EDGE_BLK, False, False, 0, EDGE_BLK)
        gran_load_dyn(i, 1)
        step(b0 + 4, 0, 0, 0, 0 * EDGE_BLK, False, False, 0, 2 * EDGE_BLK)
        step(b0 + 5, 1, 1, 0, 1 * EDGE_BLK, False, False, 0, 3 * EDGE_BLK)
        wait_gran(1)
        step(b0 + 6, 2, 0, 0, 2 * EDGE_BLK, False, False, 1, 0)
        step(b0 + 7, 3, 1, 0, 3 * EDGE_BLK, False, False, 1, EDGE_BLK)
        gran_load_dyn2(i, 0)
        return c

    def gran_load_dyn(i, P):
        sl = pl.ds(ebase + (8 * i + 12) * EDGE_BLK, GRAN)
        pltpu.async_copy(src_hbm.at[sl], sxq[P], xsem[P])
        pltpu.async_copy(w_hbm.at[sl], wvq[P], wsem[P])

    def gran_load_dyn2(i, P):
        sl = pl.ds(ebase + (8 * i + 16) * EDGE_BLK, GRAN)
        pltpu.async_copy(src_hbm.at[sl], sxq[P], xsem[P])
        pltpu.async_copy(w_hbm.at[sl], wvq[P], wsem[P])

    lax.fori_loop(0, (NQUAD - 2) // 2, _pair, 0)

    b0 = 4 * (NQUAD - 1)
    step(b0 + 0, 0, 0, 1, 0 * EDGE_BLK, False, False, 1, 2 * EDGE_BLK)
    step(b0 + 1, 1, 1, 1, 1 * EDGE_BLK, False, False, 1, 3 * EDGE_BLK)
    wait_gran(0)
    step(b0 + 2, 2, 0, 1, 2 * EDGE_BLK, False, False, 0, 0)
    step(b0 + 3, 3, 1, 1, 3 * EDGE_BLK, False, False, 0, EDGE_BLK)
    step(NBLK - 2, 0, 0, 0, 0 * EDGE_BLK, False, True)
    step(NBLK - 1, 1, 1, 0, 1 * EDGE_BLK, False, True)

    wait_rows(ssem[0], rows0)
    wait_rows(ssem[1], rows1)

    plsc.subcore_barrier()

    @pl.when(t == 0)
    def _():
        pltpu.sync_copy(acc.at[pl.ds(row0, ROWS_PER_TILE)],
                        out0_hbm.at[pl.ds(row0, ROWS_PER_TILE)])

    @pl.when(t != 0)
    def _():
        pltpu.sync_copy(acc.at[pl.ds(row0, ROWS_PER_TILE)],
                        out1_hbm.at[pl.ds(row0, ROWS_PER_TILE)])


@jax.jit
def _sc_propagate(x0, x1, srcp, dst, wp):
    mesh = plsc.VectorSubcoreMesh(core_axis_name="c", subcore_axis_name="s")
    f = pl.kernel(
        _sc_body,
        out_type=(jax.ShapeDtypeStruct((NP, D), jnp.float32),
                  jax.ShapeDtypeStruct((NP, D), jnp.float32)),
        mesh=mesh,
        scratch_types=(
            [pltpu.VMEM_SHARED((NP, D), jnp.float32)]
            + [pltpu.VMEM((EDGE_BLK, D), jnp.float32) for _ in range(4)]
            + [pltpu.VMEM((EDGE_BLK,), jnp.int32) for _ in range(4)]
            + [pltpu.VMEM((GRAN,), jnp.int32) for _ in range(2)]
            + [pltpu.VMEM((GRAN,), jnp.float32) for _ in range(2)]
            + [pltpu.SemaphoreType.DMA for _ in range(17)]
        ),
    )
    return f(x0, x1, srcp, dst, wp)


def _tc_mid_body(x0_ref, x1_ref, w_ref, o0_ref, o1_ref):
    w = w_ref[...]
    o0_ref[...] = jnp.maximum(
        jnp.dot(x0_ref[...], w, preferred_element_type=jnp.float32), 0.0)
    o1_ref[...] = jnp.maximum(
        jnp.dot(x1_ref[...], w, preferred_element_type=jnp.float32), 0.0)


@jax.jit
def _tc_mid(a0, a1, W):
    bn = 2048
    return pl.pallas_call(
        _tc_mid_body,
        grid=(NP // bn,),
        in_specs=[
            pl.BlockSpec((bn, D), lambda i: (i, 0)),
            pl.BlockSpec((bn, D), lambda i: (i, 0)),
            pl.BlockSpec((D, D), lambda i: (0, 0)),
        ],
        out_specs=[
            pl.BlockSpec((bn, D), lambda i: (i, 0)),
            pl.BlockSpec((bn, D), lambda i: (i, 0)),
        ],
        out_shape=(jax.ShapeDtypeStruct((NP, D), jnp.float32),
                   jax.ShapeDtypeStruct((NP, D), jnp.float32)),
    )(a0, a1, W)


def _tc_final_body(x0_ref, x1_ref, w_ref, o_ref):
    w = w_ref[...]
    o_ref[0] = jnp.maximum(
        jnp.dot(x0_ref[...], w, preferred_element_type=jnp.float32), 0.0)
    o_ref[1] = jnp.maximum(
        jnp.dot(x1_ref[...], w, preferred_element_type=jnp.float32), 0.0)


@jax.jit
def _tc_final(a0, a1, W):
    bn = 2000
    return pl.pallas_call(
        _tc_final_body,
        grid=(N // bn,),
        in_specs=[
            pl.BlockSpec((bn, D), lambda i: (i, 0)),
            pl.BlockSpec((bn, D), lambda i: (i, 0)),
            pl.BlockSpec((D, D), lambda i: (0, 0)),
        ],
        out_specs=pl.BlockSpec((T, bn, D), lambda i: (0, i, 0)),
        out_shape=jax.ShapeDtypeStruct((T, N, D), jnp.float32),
    )(a0, a1, W)


def kernel(node_embs, edge_index, edge_weight, W1, W2):
    src = edge_index[0]
    dst = edge_index[1]
    pad_i = jnp.zeros((EPAD,), jnp.int32)
    pad_f = jnp.zeros((EPAD,), jnp.float32)
    srcp = jnp.concatenate([src, pad_i])
    wp = jnp.concatenate([edge_weight, pad_f])
    a10, a11 = _sc_propagate(node_embs[0], node_embs[1], srcp, dst, wp)
    h10, h11 = _tc_mid(a10, a11, W1)
    a20, a21 = _sc_propagate(h10, h11, srcp, dst, wp)
    return _tc_final(a20, a21, W2)


# speedup vs baseline: 1.0415x; 1.0215x over previous
import functools

import jax
import jax.numpy as jnp
from jax import lax
from jax.experimental import pallas as pl
from jax.experimental.pallas import tpu as pltpu
from jax.experimental.pallas import tpu_sc as plsc

N = 10000
E = 320000
T = 2
D = 128

NP = 10240
NUM_TILES = 16
LANES = 16
EDGE_BLK = 80
GRAN = 4 * EDGE_BLK
E_PER_TILE = E // NUM_TILES
NBLK = E_PER_TILE // EDGE_BLK
NQUAD = NBLK // 4
ROWS_PER_TILE = NP // NUM_TILES
EPAD = GRAN // 2


def _sc_body(x0_hbm, x1_hbm, src_hbm, dst_hbm, w_hbm, out0_hbm, out1_hbm,
             acc, rows0, rows1, rows2, rows3, dx0, dx1, dx2, dx3,
             sxq0, sxq1, wvq0, wvq1,
             gsem0, gsem1, gsem2, gsem3, ssem0, ssem1, ssem2, ssem3,
             dsem0, dsem1, dsem2, dsem3,
             xsem0, xsem1, wsem0, wsem1, zsem):
    t = lax.axis_index("c")
    s = lax.axis_index("s")
    ebase = s * E_PER_TILE
    row0 = s * ROWS_PER_TILE

    rows = (rows0, rows1, rows2, rows3)
    dxb = (dx0, dx1, dx2, dx3)
    sxq = (sxq0, sxq1)
    wvq = (wvq0, wvq1)
    gsem = (gsem0, gsem1, gsem2, gsem3)
    ssem = (ssem0, ssem1, ssem2, ssem3)
    dsem = (dsem0, dsem1, dsem2, dsem3)
    xsem = (xsem0, xsem1)
    wsem = (wsem0, wsem1)

    def _zrow(r, c):
        for d in range(D // LANES):
            rows2[r, pl.ds(d * LANES, LANES)] = jnp.zeros((LANES,),
                                                          jnp.float32)
        return c
    lax.fori_loop(0, EDGE_BLK, _zrow, 0)
    for z in range(ROWS_PER_TILE // EDGE_BLK):
        pltpu.async_copy(rows2, acc.at[pl.ds(row0 + z * EDGE_BLK, EDGE_BLK)],
                         zsem)

    def didx_load(b, r4, p=0):
        pltpu.async_copy(dst_hbm.at[pl.ds(ebase + b * EDGE_BLK, EDGE_BLK)],
                         dxb[r4], dsem[r4])

    def gran_load(g, P):
        sl = pl.ds(ebase + g * GRAN, GRAN)
        pltpu.async_copy(src_hbm.at[sl], sxq[P], xsem[P])
        pltpu.async_copy(w_hbm.at[sl], wvq[P], wsem[P])

    def wait_blk(sem, dst):
        pltpu.make_async_copy(dst_hbm.at[pl.ds(0, EDGE_BLK)], dst, sem).wait()

    def wait_gran(P):
        pltpu.make_async_copy(src_hbm.at[pl.ds(0, GRAN)], sxq[P],
                              xsem[P]).wait()
        pltpu.make_async_copy(w_hbm.at[pl.ds(0, GRAN)], wvq[P],
                              wsem[P]).wait()

    def wait_rows(sem, dst):
        pltpu.make_async_copy(x0_hbm.at[pl.ds(0, EDGE_BLK)], dst, sem).wait()

    def gather(xP, xoff, r4, p=0):
        idx = sxq[xP].at[pl.ds(xoff, EDGE_BLK)]

        @pl.when(t == 0)
        def _():
            pltpu.async_copy(x0_hbm.at[idx], rows[r4], gsem[r4])

        @pl.when(t != 0)
        def _():
            pltpu.async_copy(x1_hbm.at[idx], rows[r4], gsem[r4])

    def step(b, r4, p, wP, woff, first, last, xP=0, xoff=0):
        r4n = (r4 + 2) % 4
        if not last:
            didx_load(b + 2, r4n)
            gather(xP, xoff, r4n)
        wait_rows(gsem[r4], rows[r4])
        wait_blk(dsem[r4], dxb[r4])

        def _sg(g, c):
            w16 = wvq[wP][pl.ds(woff + g * LANES, LANES)]
            for j in range(LANES):
                e = g * LANES + j
                w_e = w16[j]
                for d in range(D // LANES):
                    sl = pl.ds(d * LANES, LANES)
                    rows[r4][e, sl] = rows[r4][e, sl] * w_e
            return c
        lax.fori_loop(0, EDGE_BLK // LANES, _sg, 0)

        if not first:
            wait_rows(ssem[(r4 + 3) % 4], rows[(r4 + 3) % 4])
        pltpu.async_copy(rows[r4], acc.at[dxb[r4]], ssem[r4], add=True)

    didx_load(0, 0, 0)
    didx_load(1, 1, 1)
    gran_load(0, 0)
    wait_gran(0)
    gran_load(1, 1)
    gather(0, 0 * EDGE_BLK, 0, 0)
    gather(0, 1 * EDGE_BLK, 1, 1)
    for z in range(ROWS_PER_TILE // EDGE_BLK):
        pltpu.make_async_copy(x0_hbm.at[pl.ds(0, EDGE_BLK)], rows2,
                              zsem).wait()
    plsc.subcore_barrier()
    step(0, 0, 0, 0, 0 * EDGE_BLK, True, False, 0, 2 * EDGE_BLK)
    step(1, 1, 1, 0, 1 * EDGE_BLK, False, False, 0, 3 * EDGE_BLK)
    wait_gran(1)
    step(2, 2, 0, 0, 2 * EDGE_BLK, False, False, 1, 0)
    step(3, 3, 1, 0, 3 * EDGE_BLK, False, False, 1, EDGE_BLK)
    gran_load(2, 0)

    def _pair(i, c):
        b0 = 8 * i + 4
        step(b0 + 0, 0, 0, 1, 0 * EDGE_BLK, False, False, 1, 2 * EDGE_BLK)
        step(b0 + 1, 1, 1, 1, 1 * EDGE_BLK, False, False, 1, 3 * EDGE_BLK)
        wait_gran(0)
        step(b0 + 2, 2, 0, 1, 2 * EDGE_BLK, False, False, 0, 0)
        step(b0 + 3, 3, 1, 1, 3 * EDGE_BLK, False, False, 0, EDGE_BLK)
        gran_load_dyn(i, 1)
        step(b0 + 4, 0, 0, 0, 0 * EDGE_BLK, False, False, 0, 2 * EDGE_BLK)
        step(b0 + 5, 1, 1, 0, 1 * EDGE_BLK, False, False, 0, 3 * EDGE_BLK)
        wait_gran(1)
        step(b0 + 6, 2, 0, 0, 2 * EDGE_BLK, False, False, 1, 0)
        step(b0 + 7, 3, 1, 0, 3 * EDGE_BLK, False, False, 1, EDGE_BLK)
        gran_load_dyn2(i, 0)
        return c

    def gran_load_dyn(i, P):
        sl = pl.ds(ebase + (8 * i + 12) * EDGE_BLK, GRAN)
        pltpu.async_copy(src_hbm.at[sl], sxq[P], xsem[P])
        pltpu.async_copy(w_hbm.at[sl], wvq[P], wsem[P])

    def gran_load_dyn2(i, P):
        sl = pl.ds(ebase + (8 * i + 16) * EDGE_BLK, GRAN)
        pltpu.async_copy(src_hbm.at[sl], sxq[P], xsem[P])
        pltpu.async_copy(w_hbm.at[sl], wvq[P], wsem[P])

    lax.fori_loop(0, (NQUAD - 2) // 2, _pair, 0)

    b0 = 4 * (NQUAD - 1)
    step(b0 + 0, 0, 0, 1, 0 * EDGE_BLK, False, False, 1, 2 * EDGE_BLK)
    step(b0 + 1, 1, 1, 1, 1 * EDGE_BLK, False, False, 1, 3 * EDGE_BLK)
    wait_gran(0)
    step(b0 + 2, 2, 0, 1, 2 * EDGE_BLK, False, False, 0, 0)
    step(b0 + 3, 3, 1, 1, 3 * EDGE_BLK, False, False, 0, EDGE_BLK)
    step(NBLK - 2, 0, 0, 0, 0 * EDGE_BLK, False, True)
    step(NBLK - 1, 1, 1, 0, 1 * EDGE_BLK, False, True)

    wait_rows(ssem[1], rows1)

    plsc.subcore_barrier()

    @pl.when(t == 0)
    def _():
        pltpu.sync_copy(acc.at[pl.ds(row0, ROWS_PER_TILE)],
                        out0_hbm.at[pl.ds(row0, ROWS_PER_TILE)])

    @pl.when(t != 0)
    def _():
        pltpu.sync_copy(acc.at[pl.ds(row0, ROWS_PER_TILE)],
                        out1_hbm.at[pl.ds(row0, ROWS_PER_TILE)])


@jax.jit
def _sc_propagate(x0, x1, srcp, dst, wp):
    mesh = plsc.VectorSubcoreMesh(core_axis_name="c", subcore_axis_name="s")
    f = pl.kernel(
        _sc_body,
        out_type=(jax.ShapeDtypeStruct((NP, D), jnp.float32),
                  jax.ShapeDtypeStruct((NP, D), jnp.float32)),
        mesh=mesh,
        scratch_types=(
            [pltpu.VMEM_SHARED((NP, D), jnp.float32)]
            + [pltpu.VMEM((EDGE_BLK, D), jnp.float32) for _ in range(4)]
            + [pltpu.VMEM((EDGE_BLK,), jnp.int32) for _ in range(4)]
            + [pltpu.VMEM((GRAN,), jnp.int32) for _ in range(2)]
            + [pltpu.VMEM((GRAN,), jnp.float32) for _ in range(2)]
            + [pltpu.SemaphoreType.DMA for _ in range(17)]
        ),
    )
    return f(x0, x1, srcp, dst, wp)


def _tc_mid_body(x0_ref, x1_ref, w_ref, o0_ref, o1_ref):
    w = w_ref[...]
    o0_ref[...] = jnp.maximum(
        jnp.dot(x0_ref[...], w, preferred_element_type=jnp.float32), 0.0)
    o1_ref[...] = jnp.maximum(
        jnp.dot(x1_ref[...], w, preferred_element_type=jnp.float32), 0.0)


@jax.jit
def _tc_mid(a0, a1, W):
    bn = 2048
    return pl.pallas_call(
        _tc_mid_body,
        grid=(NP // bn,),
        in_specs=[
            pl.BlockSpec((bn, D), lambda i: (i, 0)),
            pl.BlockSpec((bn, D), lambda i: (i, 0)),
            pl.BlockSpec((D, D), lambda i: (0, 0)),
        ],
        out_specs=[
            pl.BlockSpec((bn, D), lambda i: (i, 0)),
            pl.BlockSpec((bn, D), lambda i: (i, 0)),
        ],
        out_shape=(jax.ShapeDtypeStruct((NP, D), jnp.float32),
                   jax.ShapeDtypeStruct((NP, D), jnp.float32)),
    )(a0, a1, W)


def _tc_final_body(x0_ref, x1_ref, w_ref, o_ref):
    w = w_ref[...]
    o_ref[0] = jnp.maximum(
        jnp.dot(x0_ref[...], w, preferred_element_type=jnp.float32), 0.0)
    o_ref[1] = jnp.maximum(
        jnp.dot(x1_ref[...], w, preferred_element_type=jnp.float32), 0.0)


@jax.jit
def _tc_final(a0, a1, W):
    bn = 2000
    return pl.pallas_call(
        _tc_final_body,
        grid=(N // bn,),
        in_specs=[
            pl.BlockSpec((bn, D), lambda i: (i, 0)),
            pl.BlockSpec((bn, D), lambda i: (i, 0)),
            pl.BlockSpec((D, D), lambda i: (0, 0)),
        ],
        out_specs=pl.BlockSpec((T, bn, D), lambda i: (0, i, 0)),
        out_shape=jax.ShapeDtypeStruct((T, N, D), jnp.float32),
    )(a0, a1, W)


def kernel(node_embs, edge_index, edge_weight, W1, W2):
    src = edge_index[0]
    dst = edge_index[1]
    pad_i = jnp.zeros((EPAD,), jnp.int32)
    pad_f = jnp.zeros((EPAD,), jnp.float32)
    srcp = jnp.concatenate([src, pad_i])
    wp = jnp.concatenate([edge_weight, pad_f])
    a10, a11 = _sc_propagate(node_embs[0], node_embs[1], srcp, dst, wp)
    h10, h11 = _tc_mid(a10, a11, W1)
    a20, a21 = _sc_propagate(h10, h11, srcp, dst, wp)
    return _tc_final(a20, a21, W2)
